# baseline (speedup 1.0000x reference)
import functools
import jax
import jax.numpy as jnp
from jax import lax
from jax.experimental import pallas as pl
from jax.experimental.pallas import tpu as pltpu
from jax.experimental.pallas import tpu_sc as plsc

_N = 20000
_LANES = 128
_ROWS = 160
_NP = _ROWS * _LANES
_PRE = 4096
_CROWS = _PRE // _LANES
_POST = 512
_IOU_THRESH = 0.7
_NEG = -1e30
_BIG_I = (1 << 30) - 1

_NT = 32
_EPT = _NP // _NT
_IROWS = _EPT // 128


def _prefix_exclusive(m):
    la = lax.broadcasted_iota(jnp.int32, (_LANES, _LANES), 0)
    lb = lax.broadcasted_iota(jnp.int32, (_LANES, _LANES), 1)
    upper = (la <= lb).astype(jnp.float32)
    csum = jnp.dot(m, upper, preferred_element_type=jnp.float32)
    rowtot = csum[:, _LANES - 1:_LANES]
    ra = lax.broadcasted_iota(jnp.int32, (_ROWS, _ROWS), 0)
    rb = lax.broadcasted_iota(jnp.int32, (_ROWS, _ROWS), 1)
    lstrict = (rb < ra).astype(jnp.float32)
    blockoff = jnp.dot(lstrict, rowtot, preferred_element_type=jnp.float32)
    return blockoff + csum - m


def _include_kernel(sc_ref, inc_ref):
    sc = sc_ref[...]
    keys = lax.bitcast_convert_type(sc, jnp.int32)

    def bs_body(_, lohi):
        lo, hi = lohi
        mid = lo + (hi - lo) // 2
        cnt = jnp.sum((keys >= mid).astype(jnp.int32))
        ge = cnt >= _PRE
        return (jnp.where(ge, mid, lo), jnp.where(ge, hi, mid))

    lo, _ = lax.fori_loop(0, 31, bs_body, (jnp.int32(0), jnp.int32(1 << 30)))
    c_gt = jnp.sum((keys > lo).astype(jnp.int32))
    r = (_PRE - c_gt).astype(jnp.float32)

    eqf = (keys == lo).astype(jnp.float32)
    pe = _prefix_exclusive(eqf)
    incf = (keys > lo).astype(jnp.float32) + eqf * (pe < r).astype(jnp.float32)
    pos = _prefix_exclusive(incf).astype(jnp.int32)
    inc_ref[...] = jnp.where(incf > 0, pos, _PRE)


def _compute_include(sc_plane):
    return pl.pallas_call(
        _include_kernel,
        out_shape=jax.ShapeDtypeStruct((_ROWS, _LANES), jnp.int32),
    )(sc_plane)


def _sc_scatter_body(tgt_hbm, table_hbm, out_hbm, tgt_v, rows_v, sem):
    wid = lax.axis_index("c") * 16 + lax.axis_index("s")
    pltpu.sync_copy(tgt_hbm.at[pl.ds(wid * _IROWS, _IROWS), :], tgt_v)
    pltpu.sync_copy(table_hbm.at[pl.ds(wid * _EPT, _EPT)], rows_v)
    scatters = [
        pltpu.async_copy(rows_v.at[pl.ds(i * 128, 128)],
                         out_hbm.at[tgt_v.at[i]], sem)
        for i in range(_IROWS)
    ]
    for sc_ in scatters:
        sc_.wait()


def _sc_compact(tgt_plane, table):
    kfn = pl.kernel(
        _sc_scatter_body,
        out_type=jax.ShapeDtypeStruct((_PRE + 1, 8), jnp.float32),
        mesh=plsc.VectorSubcoreMesh(core_axis_name="c", subcore_axis_name="s"),
        compiler_params=pltpu.CompilerParams(use_tc_tiling_on_sc=False),
        scratch_types=[
            pltpu.VMEM((_IROWS, 128), jnp.int32),
            pltpu.VMEM((_EPT, 8), jnp.float32),
            pltpu.SemaphoreType.DMA,
        ],
    )
    return kfn(tgt_plane, table)


def _nms4k_kernel(x1_ref, y1_ref, x2_ref, y2_ref, sc_ref, out_ref):
    x1 = x1_ref[...]
    y1 = y1_ref[...]
    x2 = x2_ref[...]
    y2 = y2_ref[...]
    work0 = sc_ref[...]
    giota = (lax.broadcasted_iota(jnp.int32, (_CROWS, _LANES), 0) * _LANES
             + lax.broadcasted_iota(jnp.int32, (_CROWS, _LANES), 1))
    area = (x2 - x1) * (y2 - y1)
    lane = lax.broadcasted_iota(jnp.int32, (1, _LANES), 1)

    def step(i, st):
        work, fb, fsc = st
        m = jnp.max(work, axis=(0, 1), keepdims=True)
        mi = jnp.min(jnp.where(work == m, giota, _BIG_I),
                     axis=(0, 1), keepdims=True)
        sel = giota == mi
        bx1 = jnp.max(jnp.where(sel, x1, _NEG), axis=(0, 1), keepdims=True)
        by1 = jnp.max(jnp.where(sel, y1, _NEG), axis=(0, 1), keepdims=True)
        bx2 = jnp.max(jnp.where(sel, x2, _NEG), axis=(0, 1), keepdims=True)
        by2 = jnp.max(jnp.where(sel, y2, _NEG), axis=(0, 1), keepdims=True)

        ix1 = jnp.maximum(bx1, x1)
        iy1 = jnp.maximum(by1, y1)
        ix2 = jnp.minimum(bx2, x2)
        iy2 = jnp.minimum(by2, y2)
        iw = jnp.maximum(ix2 - ix1, 0.0)
        ih = jnp.maximum(iy2 - iy1, 0.0)
        inter = iw * ih
        barea = (bx2 - bx1) * (by2 - by1)
        union = barea + area - inter
        iou = inter / jnp.maximum(union, 1e-8)
        work = jnp.where((iou > _IOU_THRESH) | sel, _NEG, work)

        is_first = i == 0
        fb = jnp.where(is_first, jnp.concatenate(
            [bx1, by1, bx2, by2], axis=0), fb)
        fsc = jnp.where(is_first, m, fsc)

        is_deg = m == _NEG
        ob1 = jnp.where(is_deg, fb[0:1, :], bx1)
        ob2 = jnp.where(is_deg, fb[1:2, :], by1)
        ob3 = jnp.where(is_deg, fb[2:3, :], bx2)
        ob4 = jnp.where(is_deg, fb[3:4, :], by2)
        osc = jnp.where(is_deg, fsc, m)

        rowv = jnp.where(lane == 0, ob1,
               jnp.where(lane == 1, ob2,
               jnp.where(lane == 2, ob3,
               jnp.where(lane == 3, ob4,
               jnp.where(lane == 4, osc, 0.0)))))
        out_ref[pl.ds(i, 1), :] = rowv
        return (work, fb, fsc)

    fb0 = jnp.zeros((4, 1), jnp.float32)
    fsc0 = jnp.zeros((1, 1), jnp.float32)
    lax.fori_loop(0, _POST, step, (work0, fb0, fsc0))


def _nms4k(planes):
    return pl.pallas_call(
        _nms4k_kernel,
        out_shape=jax.ShapeDtypeStruct((_POST, _LANES), jnp.float32),
    )(planes[0], planes[1], planes[2], planes[3], planes[4])


def kernel(boxes, scores):
    pad = _NP - _N
    scp = jnp.pad(scores, (0, pad), constant_values=-1.0)
    sc_plane = scp.reshape(_ROWS, _LANES)
    table = jnp.concatenate(
        [boxes, scores[:, None], jnp.zeros((_N, 3), jnp.float32)], axis=1)
    table = jnp.pad(table, ((0, pad), (0, 0)))

    tgt = _compute_include(sc_plane)
    cand = _sc_compact(tgt, table)[:_PRE, :5]
    planes = jnp.transpose(cand).reshape(5, _CROWS, _LANES)
    out = _nms4k([planes[i] for i in range(5)])
    return out[:, :5]


# speedup vs baseline: 1.1343x; 1.1343x over previous
import functools
import jax
import jax.numpy as jnp
from jax import lax
from jax.experimental import pallas as pl
from jax.experimental.pallas import tpu as pltpu
from jax.experimental.pallas import tpu_sc as plsc

_N = 20000
_LANES = 128
_ROWS = 160
_NP = _ROWS * _LANES
_PRE = 4096
_CROWS = _PRE // _LANES
_POST = 512
_IOU_THRESH = 0.7
_NEG = -1e30
_BIG_I = (1 << 30) - 1

_NT = 32
_EPT = _NP // _NT
_IROWS = _EPT // 128


def _prefix_exclusive(m):
    la = lax.broadcasted_iota(jnp.int32, (_LANES, _LANES), 0)
    lb = lax.broadcasted_iota(jnp.int32, (_LANES, _LANES), 1)
    upper = (la <= lb).astype(jnp.float32)
    csum = jnp.dot(m, upper, preferred_element_type=jnp.float32)
    rowtot = csum[:, _LANES - 1:_LANES]
    ra = lax.broadcasted_iota(jnp.int32, (_ROWS, _ROWS), 0)
    rb = lax.broadcasted_iota(jnp.int32, (_ROWS, _ROWS), 1)
    lstrict = (rb < ra).astype(jnp.float32)
    blockoff = jnp.dot(lstrict, rowtot, preferred_element_type=jnp.float32)
    return blockoff + csum - m


def _include_kernel(sc_ref, inc_ref):
    sc = sc_ref[...]
    keys = lax.bitcast_convert_type(sc, jnp.int32)

    def bs_body(_, lohi):
        lo, hi = lohi
        mid = lo + (hi - lo) // 2
        cnt = jnp.sum((keys >= mid).astype(jnp.int32))
        ge = cnt >= _PRE
        return (jnp.where(ge, mid, lo), jnp.where(ge, hi, mid))

    lo, _ = lax.fori_loop(0, 31, bs_body, (jnp.int32(0), jnp.int32(1 << 30)))
    c_gt = jnp.sum((keys > lo).astype(jnp.int32))
    r = (_PRE - c_gt).astype(jnp.float32)

    eqf = (keys == lo).astype(jnp.float32)
    pe = _prefix_exclusive(eqf)
    incf = (keys > lo).astype(jnp.float32) + eqf * (pe < r).astype(jnp.float32)
    pos = _prefix_exclusive(incf).astype(jnp.int32)
    inc_ref[...] = jnp.where(incf > 0, pos, _PRE)


def _compute_include(sc_plane):
    return pl.pallas_call(
        _include_kernel,
        out_shape=jax.ShapeDtypeStruct((_ROWS, _LANES), jnp.int32),
    )(sc_plane)


def _sc_scatter_body(tgt_hbm, table_hbm, out_hbm, tgt_v, rows_v, sem):
    wid = lax.axis_index("c") * 16 + lax.axis_index("s")
    pltpu.sync_copy(tgt_hbm.at[pl.ds(wid * _IROWS, _IROWS), :], tgt_v)
    pltpu.sync_copy(table_hbm.at[pl.ds(wid * _EPT, _EPT)], rows_v)
    scatters = [
        pltpu.async_copy(rows_v.at[pl.ds(i * 128, 128)],
                         out_hbm.at[tgt_v.at[i]], sem)
        for i in range(_IROWS)
    ]
    for sc_ in scatters:
        sc_.wait()


def _sc_compact(tgt_plane, table):
    kfn = pl.kernel(
        _sc_scatter_body,
        out_type=jax.ShapeDtypeStruct((_PRE + 1, 8), jnp.float32),
        mesh=plsc.VectorSubcoreMesh(core_axis_name="c", subcore_axis_name="s"),
        compiler_params=pltpu.CompilerParams(use_tc_tiling_on_sc=False),
        scratch_types=[
            pltpu.VMEM((_IROWS, 128), jnp.int32),
            pltpu.VMEM((_EPT, 8), jnp.float32),
            pltpu.SemaphoreType.DMA,
        ],
    )
    return kfn(tgt_plane, table)


def _nms4k_kernel(bi_ref, x1_ref, y1_ref, x2_ref, y2_ref, sc_ref, out_ref):
    x1 = x1_ref[...]
    y1 = y1_ref[...]
    x2 = x2_ref[...]
    y2 = y2_ref[...]
    work0 = sc_ref[...]
    giota = (lax.broadcasted_iota(jnp.int32, (_CROWS, _LANES), 0) * _LANES
             + lax.broadcasted_iota(jnp.int32, (_CROWS, _LANES), 1))
    area = (x2 - x1) * (y2 - y1)
    lane = lax.broadcasted_iota(jnp.int32, (1, _LANES), 1)
    lane4 = lax.broadcasted_iota(jnp.int32, (4, _LANES), 1)

    def step(i, st):
        work, fb, fsc = st
        m = jnp.max(work, axis=(0, 1), keepdims=True)
        j = jnp.min(jnp.where(work == m, giota, _BIG_I))
        row = j // _LANES
        lj = j - row * _LANES
        quad = bi_ref[pl.ds(row * 4, 4), :]
        ext = jnp.max(jnp.where(lane4 == lj, quad, _NEG),
                      axis=1, keepdims=True)

        is_first = i == 0
        fb = jnp.where(is_first, ext, fb)
        fsc = jnp.where(is_first, m, fsc)

        bx1 = ext[0:1, :]
        by1 = ext[1:2, :]
        bx2 = ext[2:3, :]
        by2 = ext[3:4, :]
        ix1 = jnp.maximum(bx1, x1)
        iy1 = jnp.maximum(by1, y1)
        ix2 = jnp.minimum(bx2, x2)
        iy2 = jnp.minimum(by2, y2)
        iw = jnp.maximum(ix2 - ix1, 0.0)
        ih = jnp.maximum(iy2 - iy1, 0.0)
        inter = iw * ih
        barea = (bx2 - bx1) * (by2 - by1)
        union = barea + area - inter
        iou = inter / jnp.maximum(union, 1e-8)
        suppress = (iou > _IOU_THRESH) | (giota == j)
        work = jnp.where(suppress, _NEG, work)

        is_deg = m == _NEG
        outb = jnp.where(is_deg, fb, ext)
        osc = jnp.where(is_deg, fsc, m)

        rowv = jnp.where(lane == 0, outb[0:1, :],
               jnp.where(lane == 1, outb[1:2, :],
               jnp.where(lane == 2, outb[2:3, :],
               jnp.where(lane == 3, outb[3:4, :],
               jnp.where(lane == 4, osc, 0.0)))))
        out_ref[pl.ds(i, 1), :] = rowv
        return (work, fb, fsc)

    fb0 = jnp.zeros((4, 1), jnp.float32)
    fsc0 = jnp.zeros((1, 1), jnp.float32)
    lax.fori_loop(0, _POST, step, (work0, fb0, fsc0))


def _nms4k(binter, planes):
    return pl.pallas_call(
        _nms4k_kernel,
        out_shape=jax.ShapeDtypeStruct((_POST, _LANES), jnp.float32),
    )(binter, planes[0], planes[1], planes[2], planes[3], planes[4])


def kernel(boxes, scores):
    pad = _NP - _N
    scp = jnp.pad(scores, (0, pad), constant_values=-1.0)
    sc_plane = scp.reshape(_ROWS, _LANES)
    table = jnp.concatenate(
        [boxes, scores[:, None], jnp.zeros((_N, 3), jnp.float32)], axis=1)
    table = jnp.pad(table, ((0, pad), (0, 0)))

    tgt = _compute_include(sc_plane)
    cand = _sc_compact(tgt, table)[:_PRE, :5]
    planes = jnp.transpose(cand).reshape(5, _CROWS, _LANES)
    binter = jnp.transpose(planes[:4], (1, 0, 2)).reshape(4 * _CROWS, _LANES)
    out = _nms4k(binter, [planes[i] for i in range(5)])
    return out[:, :5]
